# baseline (speedup 1.0000x reference)
import math

import jax
import jax.numpy as jnp
from jax import lax
from jax.experimental import pallas as pl
from jax.experimental.pallas import tpu as pltpu
from jax.experimental.pallas import tpu_sc as plsc

_HALF_PI = math.pi / 2
_NC, _NS, _LANES = 2, 16, 16
_NW = _NC * _NS
_NBUF = 3
_PADDED = 128


def _make_lookup(b0: int, b1: int, dim: int):
    assert b0 % _NW == 0
    blk = b0 // _NW
    assert blk % _LANES == 0
    vecs_per_blk = blk // _LANES
    nbuf = _NBUF

    mesh = plsc.VectorSubcoreMesh(core_axis_name="c", subcore_axis_name="s")

    def body(xt_hbm, w_hbm, o2_hbm, idx_v, *bufs_and_sems):
        g = bufs_and_sems[:nbuf]
        tb = bufs_and_sems[nbuf:2 * nbuf]
        gsems = bufs_and_sems[2 * nbuf:3 * nbuf]
        ssems = bufs_and_sems[3 * nbuf:4 * nbuf]

        wid = lax.axis_index("s") * _NC + lax.axis_index("c")
        base = wid * blk
        pltpu.sync_copy(xt_hbm.at[:, pl.ds(base, blk)], idx_v)

        def gather(j, b):
            return pltpu.make_async_copy(
                w_hbm.at[idx_v.at[j]], g[b], gsems[b])

        def store(j, b):
            return pltpu.make_async_copy(
                tb[b], o2_hbm.at[j, :, pl.ds(base, blk)], ssems[b])

        rowvecs = [
            lax.iota(jnp.int32, _LANES) + v * _LANES
            for v in range(vecs_per_blk)
        ]

        def transpose_scale(b):
            src, dst = g[b], tb[b]

            @plsc.parallel_loop(0, dim, unroll=2)
            def _(d):
                col = jnp.full((_LANES,), d, jnp.int32)
                for v in range(vecs_per_blk):
                    vals = plsc.load_gather(src, [rowvecs[v], col])
                    dst[d, pl.ds(v * _LANES, _LANES)] = vals * _HALF_PI

        def step(j, b):
            gather(j, b).wait()
            transpose_scale(b)
            store(j, b).start()

        for b in range(nbuf):
            gather(b, b).start()

        if b1 > nbuf:
            def ring(gi, carry):
                for b in range(nbuf):
                    j = gi * nbuf + b
                    step(j, b)
                    store(j, b).wait()
                    gather(j + nbuf, b).start()
                return carry

            nfull = (b1 - nbuf) // nbuf
            lax.fori_loop(0, nfull, ring, 0)
            for k in range(nfull * nbuf, b1 - nbuf):
                b = k % nbuf
                step(k, b)
                store(k, b).wait()
                gather(k + nbuf, b).start()

        for k in range(b1 - nbuf, b1):
            b = k % nbuf
            step(k, b)
        for k in range(b1 - nbuf, b1):
            store(k, k % nbuf).wait()

    scratch = [pltpu.VMEM((b1, blk), jnp.int32)]
    scratch += [pltpu.VMEM((blk, _PADDED), jnp.float32) for _ in range(nbuf)]
    scratch += [pltpu.VMEM((dim, blk), jnp.float32) for _ in range(nbuf)]
    scratch += [pltpu.SemaphoreType.DMA for _ in range(2 * nbuf)]

    return pl.kernel(
        body,
        out_type=jax.ShapeDtypeStruct((b1, dim, b0), jnp.float32),
        mesh=mesh,
        scratch_types=scratch,
        compiler_params=pltpu.CompilerParams(
            use_tc_tiling_on_sc=True, needs_layout_passes=False),
    )


def kernel(x, weight):
    b0, b1 = x.shape
    n, dim = weight.shape
    wp = jnp.pad(weight, ((0, 0), (0, _PADDED - dim)))
    xt = jnp.swapaxes(x.astype(jnp.int32), 0, 1)
    o2 = _make_lookup(b0, b1, dim)(xt, wp)
    return jnp.transpose(o2, (2, 0, 1))


# speedup vs baseline: 1.0426x; 1.0426x over previous
import math

import jax
import jax.numpy as jnp
from jax import lax
from jax.experimental import pallas as pl
from jax.experimental.pallas import tpu as pltpu
from jax.experimental.pallas import tpu_sc as plsc

_HALF_PI = math.pi / 2
_NC, _NS, _LANES = 2, 16, 16
_NW = _NC * _NS
_NBUF = 8
_PADDED = 128


def _make_lookup(b0: int, b1: int, dim: int):
    assert b0 % _NW == 0
    rows_w = b0 // _NW
    nchunk = rows_w
    nbuf = _NBUF
    while nchunk % nbuf or nchunk <= nbuf:
        nbuf //= 2
    vecs_per_row = dim // _LANES
    assert dim % _LANES == 0

    mesh = plsc.VectorSubcoreMesh(core_axis_name="c", subcore_axis_name="s")

    def body(x_hbm, w_hbm, out_hbm, idx_v, *bufs_and_sems):
        g = bufs_and_sems[:nbuf]
        ob = bufs_and_sems[nbuf:2 * nbuf]
        gsems = bufs_and_sems[2 * nbuf:3 * nbuf]
        ssems = bufs_and_sems[3 * nbuf:4 * nbuf]

        wid = lax.axis_index("s") * _NC + lax.axis_index("c")
        base = wid * rows_w
        pltpu.sync_copy(x_hbm.at[pl.ds(base, rows_w), :], idx_v)

        def gather(c, b):
            return pltpu.make_async_copy(
                w_hbm.at[idx_v.at[c]], g[b], gsems[b])

        def store(c, b):
            return pltpu.make_async_copy(
                ob[b], out_hbm.at[base + c], ssems[b])

        def scale(b):
            src, dst = g[b], ob[b]

            @plsc.parallel_loop(0, b1, unroll=13)
            def _(k):
                for j in range(vecs_per_row):
                    sl = pl.ds(j * _LANES, _LANES)
                    dst[k, sl] = src[k, sl] * _HALF_PI

        def step(c, b):
            gather(c, b).wait()
            scale(b)
            store(c, b).start()

        for b in range(nbuf):
            gather(b, b).start()

        if nchunk > nbuf:
            def outer(gi, carry):
                for b in range(nbuf):
                    c = gi * nbuf + b
                    step(c, b)
                    store(c, b).wait()
                    gather(c + nbuf, b).start()
                return carry

            lax.fori_loop(0, nchunk // nbuf - 1, outer, 0)

        for b in range(nbuf):
            c = nchunk - nbuf + b
            step(c, b)
        for b in range(nbuf):
            store(nchunk - nbuf + b, b).wait()

    scratch = [pltpu.VMEM((rows_w, b1), jnp.int32)]
    scratch += [pltpu.VMEM((b1, _PADDED), jnp.float32) for _ in range(nbuf)]
    scratch += [pltpu.VMEM((b1, dim), jnp.float32) for _ in range(nbuf)]
    scratch += [pltpu.SemaphoreType.DMA for _ in range(2 * nbuf)]

    return pl.kernel(
        body,
        out_type=jax.ShapeDtypeStruct((b0, b1, dim), jnp.float32),
        mesh=mesh,
        scratch_types=scratch,
        compiler_params=pltpu.CompilerParams(use_tc_tiling_on_sc=True),
    )


def kernel(x, weight):
    b0, b1 = x.shape
    n, dim = weight.shape
    wp = jnp.pad(weight, ((0, 0), (0, _PADDED - dim)))
    return _make_lookup(b0, b1, dim)(x.astype(jnp.int32), wp)


# speedup vs baseline: 1.2875x; 1.2348x over previous
import math

import jax
import jax.numpy as jnp
from jax import lax
from jax.experimental import pallas as pl
from jax.experimental.pallas import tpu as pltpu
from jax.experimental.pallas import tpu_sc as plsc

_HALF_PI = math.pi / 2
_NC, _NS, _LANES = 2, 16, 16
_NW = _NC * _NS
_NBUF = 8
_PADDED = 128


def _make_lookup(b0: int, b1: int, dim: int):
    assert b0 % _NW == 0
    rows_w = b0 // _NW
    nchunk = rows_w
    nbuf = _NBUF
    while nchunk % nbuf or nchunk <= nbuf:
        nbuf //= 2
    vecs_per_row = dim // _LANES
    assert dim % _LANES == 0

    mesh = plsc.VectorSubcoreMesh(core_axis_name="c", subcore_axis_name="s")

    def body(x_hbm, w_hbm, out_hbm, idx_v, *bufs_and_sems):
        g = bufs_and_sems[:nbuf]
        ob = bufs_and_sems[nbuf:2 * nbuf]
        gsems = bufs_and_sems[2 * nbuf:3 * nbuf]
        ssems = bufs_and_sems[3 * nbuf:4 * nbuf]

        wid = lax.axis_index("s") * _NC + lax.axis_index("c")
        base = wid * rows_w
        pltpu.sync_copy(x_hbm.at[pl.ds(base, rows_w), :], idx_v)

        def gather(c, b):
            return pltpu.make_async_copy(
                w_hbm.at[idx_v.at[c]], g[b], gsems[b])

        def store(c, b):
            return pltpu.make_async_copy(
                ob[b], out_hbm.at[base + c], ssems[b])

        def scale(b):
            src, dst = g[b], ob[b]

            @plsc.parallel_loop(0, b1, unroll=2)
            def _(k):
                for j in range(vecs_per_row):
                    dst[pl.ds(k * dim + j * _LANES, _LANES)] = (
                        src[k, pl.ds(j * _LANES, _LANES)] * _HALF_PI)

        def step(c, b):
            gather(c, b).wait()
            scale(b)
            store(c, b).start()

        for b in range(nbuf):
            gather(b, b).start()

        if nchunk > nbuf:
            def outer(gi, carry):
                for b in range(nbuf):
                    c = gi * nbuf + b
                    step(c, b)
                    store(c, b).wait()
                    gather(c + nbuf, b).start()
                return carry

            lax.fori_loop(0, nchunk // nbuf - 1, outer, 0)

        for b in range(nbuf):
            c = nchunk - nbuf + b
            step(c, b)
        for b in range(nbuf):
            store(nchunk - nbuf + b, b).wait()

    scratch = [pltpu.VMEM((rows_w, b1), jnp.int32)]
    scratch += [pltpu.VMEM((b1, _PADDED), jnp.float32) for _ in range(nbuf)]
    scratch += [pltpu.VMEM((b1 * dim,), jnp.float32) for _ in range(nbuf)]
    scratch += [pltpu.SemaphoreType.DMA for _ in range(2 * nbuf)]

    return pl.kernel(
        body,
        out_type=jax.ShapeDtypeStruct((b0, b1 * dim), jnp.float32),
        mesh=mesh,
        scratch_types=scratch,
        compiler_params=pltpu.CompilerParams(use_tc_tiling_on_sc=True),
    )


def kernel(x, weight):
    b0, b1 = x.shape
    n, dim = weight.shape
    wp = jnp.pad(weight, ((0, 0), (0, _PADDED - dim)))
    flat = _make_lookup(b0, b1, dim)(x.astype(jnp.int32), wp)
    return flat.reshape(b0, b1, dim)
